# baseline (speedup 1.0000x reference)
import functools

import jax
import jax.numpy as jnp
from jax.experimental import pallas as pl
from jax.experimental.pallas import tpu as pltpu


def _fused_body(x_ref, w1_ref, b1_ref, w2_ref, b2_ref, cb_ref,
                cw_ref, vq_ref, h_ref, *, nb):
    k = pl.program_id(0)

    @pl.when(k == 0)
    def _compute_h():
        for b in range(nb):
            ht = jax.lax.dot_general(
                w1_ref[...], x_ref[b],
                (((1,), (0,)), ((), ())),
                preferred_element_type=jnp.float32,
            ) + b1_ref[...]
            h_ref[b] = jnp.maximum(ht, 0.0)

    for b in range(nb):
        cw = jax.lax.dot_general(
            h_ref[b], w2_ref[...],
            (((0,), (1,)), ((), ())),
            preferred_element_type=jnp.float32,
        ) + b2_ref[...]
        cw_ref[b] = cw

        contrib = jax.lax.dot_general(
            cb_ref[...], cw,
            (((0,), (1,)), ((), ())),
            preferred_element_type=jnp.float32,
        )

        @pl.when(k == 0)
        def _init_acc():
            vq_ref[b] = contrib

        @pl.when(k > 0)
        def _acc():
            vq_ref[b] += contrib


@functools.partial(jax.jit, static_argnames=("kb",))
def _fused(x, w1, b1, w2, b2, cb, kb=512):
    nb, c, hw = x.shape
    mid = w1.shape[0]
    kk = w2.shape[0]
    grid = (kk // kb,)
    cw, vq = pl.pallas_call(
        functools.partial(_fused_body, nb=nb),
        grid=grid,
        in_specs=[
            pl.BlockSpec((nb, c, hw), lambda k: (0, 0, 0)),
            pl.BlockSpec((mid, c), lambda k: (0, 0)),
            pl.BlockSpec((mid, 1), lambda k: (0, 0)),
            pl.BlockSpec((kb, mid), lambda k: (k, 0)),
            pl.BlockSpec((1, kb), lambda k: (0, k)),
            pl.BlockSpec((kb, c), lambda k: (k, 0)),
        ],
        out_specs=[
            pl.BlockSpec((nb, hw, kb), lambda k: (0, 0, k)),
            pl.BlockSpec((nb, c, hw), lambda k: (0, 0, 0)),
        ],
        out_shape=[
            jax.ShapeDtypeStruct((nb, hw, kk), jnp.float32),
            jax.ShapeDtypeStruct((nb, c, hw), jnp.float32),
        ],
        scratch_shapes=[pltpu.VMEM((nb, mid, hw), jnp.float32)],
        compiler_params=pltpu.CompilerParams(
            dimension_semantics=("arbitrary",),
        ),
    )(x, w1, b1, w2, b2, cb)
    return cw, vq


def kernel(embed, W1, b1, W2, b2, codebook):
    Bx, Cx, Hx, Wx = embed.shape
    x = embed.reshape(Bx, Cx, Hx * Wx)
    cw, vq = _fused(x, W1, b1.reshape(-1, 1), W2, b2.reshape(1, -1), codebook)
    code_weight = cw
    embed_vq = vq.reshape(Bx, Cx, Hx, Wx)
    return (embed_vq, code_weight, codebook)


# speedup vs baseline: 1.2218x; 1.2218x over previous
import functools

import jax
import jax.numpy as jnp
from jax.experimental import pallas as pl
from jax.experimental.pallas import tpu as pltpu


def _fused_body(x_ref, w1_ref, b1_ref, w2_ref, b2_ref, cb_ref,
                cw_ref, vq_ref, h_ref, acc_ref, *, nb):
    k = pl.program_id(0)

    @pl.when(k == 0)
    def _compute_h():
        for b in range(nb):
            h = jax.lax.dot_general(
                x_ref[b], w1_ref[...],
                (((0,), (1,)), ((), ())),
                preferred_element_type=jnp.float32,
            ) + b1_ref[...]
            h_ref[b] = jnp.maximum(h, 0.0)

    for b in range(nb):
        cw = jax.lax.dot_general(
            h_ref[b], w2_ref[...],
            (((1,), (1,)), ((), ())),
            preferred_element_type=jnp.float32,
        ) + b2_ref[...]
        cw_ref[b] = cw

        contrib = jnp.dot(cw, cb_ref[...], preferred_element_type=jnp.float32)

        @pl.when(k == 0)
        def _init_acc():
            acc_ref[b] = contrib

        @pl.when(k > 0)
        def _acc():
            acc_ref[b] += contrib

    @pl.when(k == pl.num_programs(0) - 1)
    def _writeback():
        for b in range(nb):
            vq_ref[b] = acc_ref[b].T


@functools.partial(jax.jit, static_argnames=("kb",))
def _fused(x, w1, b1, w2, b2, cb, kb=512):
    nb, c, hw = x.shape
    mid = w1.shape[0]
    kk = w2.shape[0]
    grid = (kk // kb,)
    cw, vq = pl.pallas_call(
        functools.partial(_fused_body, nb=nb),
        grid=grid,
        in_specs=[
            pl.BlockSpec((nb, c, hw), lambda k: (0, 0, 0)),
            pl.BlockSpec((mid, c), lambda k: (0, 0)),
            pl.BlockSpec((1, mid), lambda k: (0, 0)),
            pl.BlockSpec((kb, mid), lambda k: (k, 0)),
            pl.BlockSpec((1, kb), lambda k: (0, k)),
            pl.BlockSpec((kb, c), lambda k: (k, 0)),
        ],
        out_specs=[
            pl.BlockSpec((nb, hw, kb), lambda k: (0, 0, k)),
            pl.BlockSpec((nb, c, hw), lambda k: (0, 0, 0)),
        ],
        out_shape=[
            jax.ShapeDtypeStruct((nb, hw, kk), jnp.float32),
            jax.ShapeDtypeStruct((nb, c, hw), jnp.float32),
        ],
        scratch_shapes=[
            pltpu.VMEM((nb, hw, mid), jnp.float32),
            pltpu.VMEM((nb, hw, c), jnp.float32),
        ],
        compiler_params=pltpu.CompilerParams(
            dimension_semantics=("arbitrary",),
        ),
    )(x, w1, b1, w2, b2, cb)
    return cw, vq


def kernel(embed, W1, b1, W2, b2, codebook):
    Bx, Cx, Hx, Wx = embed.shape
    x = embed.reshape(Bx, Cx, Hx * Wx)
    cw, vq = _fused(x, W1, b1.reshape(1, -1), W2, b2.reshape(1, -1), codebook)
    embed_vq = vq.reshape(Bx, Cx, Hx, Wx)
    return (embed_vq, cw, codebook)


# speedup vs baseline: 1.5752x; 1.2893x over previous
import functools

import jax
import jax.numpy as jnp
from jax.experimental import pallas as pl
from jax.experimental.pallas import tpu as pltpu


def _fused_body(x_ref, w1_ref, b1_ref, w2_ref, b2_ref, cb_ref,
                cw_ref, vq_ref, h_ref, *, nb, hw):
    k = pl.program_id(0)

    @pl.when(k == 0)
    def _compute_h():
        for b in range(nb):
            h = jax.lax.dot_general(
                x_ref[b], w1_ref[...],
                (((0,), (1,)), ((), ())),
                preferred_element_type=jnp.float32,
            ) + b1_ref[...]
            h_ref[pl.ds(b * hw, hw), :] = jnp.maximum(h, 0.0)

    cw = jax.lax.dot_general(
        h_ref[...], w2_ref[...],
        (((1,), (1,)), ((), ())),
        preferred_element_type=jnp.float32,
    ) + b2_ref[...]
    cw_ref[...] = cw

    contrib = jnp.dot(cw, cb_ref[...], preferred_element_type=jnp.float32)

    @pl.when(k == 0)
    def _init_acc():
        vq_ref[...] = contrib

    @pl.when(k > 0)
    def _acc():
        vq_ref[...] += contrib


@functools.partial(jax.jit, static_argnames=("kb",))
def _fused(x, w1, b1, w2, b2, cb, kb=512):
    nb, c, hw = x.shape
    n = nb * hw
    mid = w1.shape[0]
    kk = w2.shape[0]
    grid = (kk // kb,)
    cw, vq = pl.pallas_call(
        functools.partial(_fused_body, nb=nb, hw=hw),
        grid=grid,
        in_specs=[
            pl.BlockSpec((nb, c, hw), lambda k: (0, 0, 0)),
            pl.BlockSpec((mid, c), lambda k: (0, 0)),
            pl.BlockSpec((1, mid), lambda k: (0, 0)),
            pl.BlockSpec((kb, mid), lambda k: (k, 0)),
            pl.BlockSpec((1, kb), lambda k: (0, k)),
            pl.BlockSpec((kb, c), lambda k: (k, 0)),
        ],
        out_specs=[
            pl.BlockSpec((n, kb), lambda k: (0, k)),
            pl.BlockSpec((n, c), lambda k: (0, 0)),
        ],
        out_shape=[
            jax.ShapeDtypeStruct((n, kk), jnp.float32),
            jax.ShapeDtypeStruct((n, c), jnp.float32),
        ],
        scratch_shapes=[pltpu.VMEM((n, mid), jnp.float32)],
        compiler_params=pltpu.CompilerParams(
            dimension_semantics=("arbitrary",),
        ),
    )(x, w1, b1, w2, b2, cb)
    return cw, vq


def kernel(embed, W1, b1, W2, b2, codebook):
    Bx, Cx, Hx, Wx = embed.shape
    x = embed.reshape(Bx, Cx, Hx * Wx)
    cw, vq = _fused(x, W1, b1.reshape(1, -1), W2, b2.reshape(1, -1), codebook)
    code_weight = cw.reshape(Bx, Hx * Wx, -1)
    embed_vq = jnp.transpose(vq.reshape(Bx, Hx, Wx, Cx), (0, 3, 1, 2))
    return (embed_vq, code_weight, codebook)


# speedup vs baseline: 1.8707x; 1.1876x over previous
import functools

import jax
import jax.numpy as jnp
from jax.experimental import pallas as pl
from jax.experimental.pallas import tpu as pltpu


def _fused_body(x_ref, w1_ref, b1_ref, w2_ref, b2_ref, cb_ref,
                cw_ref, vq_ref, h_ref):
    k = pl.program_id(0)

    @pl.when(k == 0)
    def _compute_h():
        h = jax.lax.dot_general(
            x_ref[...], w1_ref[...],
            (((1,), (1,)), ((), ())),
            preferred_element_type=jnp.float32,
        ) + b1_ref[...]
        h_ref[...] = jnp.maximum(h, 0.0)

    cw = jax.lax.dot_general(
        h_ref[...], w2_ref[...],
        (((1,), (1,)), ((), ())),
        preferred_element_type=jnp.float32,
    ) + b2_ref[...]
    cw_ref[...] = cw

    contrib = jnp.dot(cw, cb_ref[...], preferred_element_type=jnp.float32)

    @pl.when(k == 0)
    def _init_acc():
        vq_ref[...] = contrib

    @pl.when(k > 0)
    def _acc():
        vq_ref[...] += contrib


@functools.partial(jax.jit, static_argnames=("kb",))
def _fused(x, w1, b1, w2, b2, cb, kb=1024):
    n, c = x.shape
    mid = w1.shape[0]
    kk = w2.shape[0]
    grid = (kk // kb,)
    cw, vq = pl.pallas_call(
        _fused_body,
        grid=grid,
        in_specs=[
            pl.BlockSpec((n, c), lambda k: (0, 0)),
            pl.BlockSpec((mid, c), lambda k: (0, 0)),
            pl.BlockSpec((1, mid), lambda k: (0, 0)),
            pl.BlockSpec((kb, mid), lambda k: (k, 0)),
            pl.BlockSpec((1, kb), lambda k: (0, k)),
            pl.BlockSpec((kb, c), lambda k: (k, 0)),
        ],
        out_specs=[
            pl.BlockSpec((n, kb), lambda k: (0, k)),
            pl.BlockSpec((n, c), lambda k: (0, 0)),
        ],
        out_shape=[
            jax.ShapeDtypeStruct((n, kk), jnp.float32),
            jax.ShapeDtypeStruct((n, c), jnp.float32),
        ],
        scratch_shapes=[pltpu.VMEM((n, mid), jnp.float32)],
        compiler_params=pltpu.CompilerParams(
            dimension_semantics=("arbitrary",),
        ),
    )(x, w1, b1, w2, b2, cb)
    return cw, vq


def kernel(embed, W1, b1, W2, b2, codebook):
    Bx, Cx, Hx, Wx = embed.shape
    n = Bx * Hx * Wx
    x = jnp.transpose(embed, (0, 2, 3, 1)).reshape(n, Cx)
    cw, vq = _fused(x, W1, b1.reshape(1, -1), W2, b2.reshape(1, -1), codebook)
    code_weight = cw.reshape(Bx, Hx * Wx, -1)
    embed_vq = jnp.transpose(vq.reshape(Bx, Hx, Wx, Cx), (0, 3, 1, 2))
    return (embed_vq, code_weight, codebook)
